# baseline (speedup 1.0000x reference)
import functools

import jax
import jax.numpy as jnp
from jax import lax
from jax.experimental import pallas as pl
from jax.experimental.pallas import tpu as pltpu
from jax.experimental.pallas import tpu_sc as plsc

N = 10000
E = 320000
H = 128
B = 64

NC = 2
NS = 16
LANES = 16

NW = NC * NS
CHUNK = 80
NCHUNK = 125
RBUF = 4
IBUF = 8
UNROLL = 8
MAIN_CHUNKS = 120
ROWS_PER_TILE = 624
NVREG = H // LANES
GROUPS = CHUNK // LANES


def _sc_edge_layer(h_hbm, idx_hbm, we_hbm, be_hbm, out_hbm,
                   rb0, rb1, rb2, rb3,
                   ib0, ib1, ib2, ib3, ib4, ib5, ib6, ib7,
                   we_v, be_v, agg_sp,
                   gs0, gs1, gs2, gs3, ss0, ss1, ss2, ss3,
                   is0, is1, is2, is3, is4, is5, is6, is7):
    cid = lax.axis_index("c")
    sid = lax.axis_index("s")
    wid = sid * NC + cid
    rbufs = (rb0, rb1, rb2, rb3)
    ibufs = (ib0, ib1, ib2, ib3, ib4, ib5, ib6, ib7)
    gsem = (gs0, gs1, gs2, gs3)
    ssem = (ss0, ss1, ss2, ss3)
    isem = (is0, is1, is2, is3, is4, is5, is6, is7)

    def zrow(j, _):
        zv = jnp.zeros((LANES,), jnp.float32)
        for k in range(NVREG):
            rb0[j, pl.ds(k * LANES, LANES)] = zv
        return 0
    lax.fori_loop(0, CHUNK, zrow, 0)
    for z in range(7):
        pltpu.sync_copy(rb0,
                        agg_sp.at[pl.ds(sid * ROWS_PER_TILE + z * CHUNK,
                                        CHUNK)])
    pltpu.sync_copy(rb0.at[pl.ds(0, ROWS_PER_TILE - 7 * CHUNK)],
                    agg_sp.at[pl.ds(sid * ROWS_PER_TILE + 7 * CHUNK,
                                    ROWS_PER_TILE - 7 * CHUNK)])

    @pl.when(sid == NS - 1)
    def _zero_tail():
        pltpu.sync_copy(rb0.at[pl.ds(0, N - NS * ROWS_PER_TILE)],
                        agg_sp.at[pl.ds(NS * ROWS_PER_TILE,
                                        N - NS * ROWS_PER_TILE)])

    pltpu.sync_copy(we_hbm, we_v)
    pltpu.sync_copy(be_hbm, be_v)
    wek = [we_v[pl.ds(k * LANES, LANES)] for k in range(NVREG)]
    bek = [be_v[pl.ds(k * LANES, LANES)] for k in range(NVREG)]
    plsc.subcore_barrier()

    def start_idx(i, k):
        pltpu.async_copy(idx_hbm.at[wid, i], ibufs[k], isem[k])

    def wait_idx(i, k):
        pltpu.make_async_copy(idx_hbm.at[wid, i], ibufs[k], isem[k]).wait()

    def start_gather(k, b):
        pltpu.async_copy(h_hbm.at[ibufs[k].at[0]], rbufs[b], gsem[b])

    def wait_gather(k, b):
        pltpu.make_async_copy(h_hbm.at[ibufs[k].at[0]], rbufs[b],
                              gsem[b]).wait()

    def start_scatter(k, b):
        pltpu.async_copy(rbufs[b], agg_sp.at[ibufs[k].at[1]], ssem[b],
                         add=True)

    def wait_scatter(k, b):
        pltpu.make_async_copy(rbufs[b], agg_sp.at[ibufs[k].at[1]],
                              ssem[b]).wait()

    def compute(k, b):
        def group_body(gg, _):
            ev = lax.bitcast_convert_type(
                ibufs[k][2, pl.ds(gg * LANES, LANES)], jnp.float32)
            for jj in range(LANES):
                ej = jnp.full((LANES,), ev[jj], jnp.float32)
                row = gg * LANES + jj
                for kk in range(NVREG):
                    sl = pl.ds(kk * LANES, LANES)
                    rbufs[b][row, sl] = jnp.maximum(
                        rbufs[b][row, sl] + (ej * wek[kk] + bek[kk]), 0.0)
            return 0
        lax.fori_loop(0, GROUPS, group_body, 0)

    for i in range(4):
        start_idx(i, i)
    wait_idx(0, 0)
    start_gather(0, 0)
    wait_idx(1, 1)
    start_gather(1, 1)

    def main_body(t, _):
        for u in range(UNROLL):
            i = t * UNROLL + u
            kb = u
            rb = u % RBUF

            start_idx(i + 4, (u + 4) % IBUF)
            wait_idx(i + 2, (u + 2) % IBUF)

            @pl.when(i >= 2)
            def _drain():
                wait_scatter((u - 2) % IBUF, (u + 2) % RBUF)
            start_gather((u + 2) % IBUF, (u + 2) % RBUF)
            wait_gather(kb, rb)
            compute(kb, rb)
            start_scatter(kb, rb)
        return 0
    lax.fori_loop(0, MAIN_CHUNKS // UNROLL, main_body, 0)

    for i in range(MAIN_CHUNKS, NCHUNK):
        u = i % IBUF
        rb = i % RBUF
        if i + 4 < NCHUNK:
            start_idx(i + 4, (u + 4) % IBUF)
        if i + 2 < NCHUNK:
            wait_idx(i + 2, (u + 2) % IBUF)
            wait_scatter((u - 2) % IBUF, (u + 2) % RBUF)
            start_gather((u + 2) % IBUF, (u + 2) % RBUF)
        wait_gather(u, rb)
        compute(u, rb)
        start_scatter(u, rb)
    for i in range(NCHUNK - RBUF, NCHUNK):
        wait_scatter(i % IBUF, i % RBUF)
    plsc.subcore_barrier()

    pltpu.sync_copy(agg_sp.at[pl.ds(sid * ROWS_PER_TILE, ROWS_PER_TILE)],
                    out_hbm.at[cid, pl.ds(sid * ROWS_PER_TILE, ROWS_PER_TILE)])

    @pl.when(sid == NS - 1)
    def _write_tail():
        pltpu.sync_copy(agg_sp.at[pl.ds(NS * ROWS_PER_TILE,
                                        N - NS * ROWS_PER_TILE)],
                        out_hbm.at[cid, pl.ds(NS * ROWS_PER_TILE,
                                              N - NS * ROWS_PER_TILE)])


_sc_edge_call = functools.partial(
    pl.kernel,
    out_type=jax.ShapeDtypeStruct((NC, N, H), jnp.float32),
    mesh=plsc.VectorSubcoreMesh(core_axis_name="c", subcore_axis_name="s"),
    scratch_types=(
        [pltpu.VMEM((CHUNK, H), jnp.float32)] * RBUF
        + [pltpu.VMEM((3, CHUNK), jnp.int32)] * IBUF
        + [pltpu.VMEM((H,), jnp.float32)] * 2
        + [pltpu.VMEM_SHARED((N, H), jnp.float32)]
        + [pltpu.SemaphoreType.DMA] * (RBUF + RBUF + IBUF)
    ),
)(_sc_edge_layer)


def _embed_body(x_ref, w_ref, b_ref, o_ref):
    o_ref[...] = jnp.maximum(
        jnp.dot(x_ref[...], w_ref[...], preferred_element_type=jnp.float32)
        + b_ref[...], 0.0)


def _layer_body(h_ref, agg_ref, w_ref, b_ref, o_ref):
    hh = h_ref[...] + agg_ref[0] + agg_ref[1]
    u = jnp.dot(hh, w_ref[...], preferred_element_type=jnp.float32) + b_ref[...]
    u = jnp.maximum(u, 0.0)
    u = u - jnp.mean(u, axis=0, keepdims=True)
    s = lax.rsqrt(1e-6 + jnp.mean(jnp.sum(u * u, axis=-1)))
    o_ref[...] = u * s


def _pool_body(h_ref, batch_ref, w1_ref, b1_ref, w2_ref, b2_ref, o_ref, g_ref):
    h = h_ref[...]
    bvec = batch_ref[...]

    def seg(b, _):
        mask = bvec == b
        g_ref[b, :] = jnp.max(jnp.where(mask, h, -jnp.inf), axis=0)
        return 0
    lax.fori_loop(0, B, seg, 0)
    g = g_ref[...]
    u = jnp.maximum(
        jnp.dot(g, w1_ref[...], preferred_element_type=jnp.float32)
        + b1_ref[...], 0.0)
    o_ref[...] = jnp.dot(u, w2_ref[...],
                         preferred_element_type=jnp.float32) + b2_ref[...]


def kernel(x, edge_index, edge_attr, batch, W_emb, b_emb, W_nn1, b_nn1, W_e1,
           b_e1, W_nn2, b_nn2, W_e2, b_e2, W_nn3, b_nn3, W_e3, b_e3, W_l1,
           b_l1, W_l2, b_l2):
    src = edge_index[0].reshape(NW, NCHUNK, CHUNK)
    dst = edge_index[1].reshape(NW, NCHUNK, CHUNK)
    e = lax.bitcast_convert_type(edge_attr[:, 0], jnp.int32).reshape(
        NW, NCHUNK, CHUNK)
    idx3 = jnp.stack([src, dst, e], axis=2)

    h = pl.pallas_call(
        _embed_body,
        out_shape=jax.ShapeDtypeStruct((N, H), jnp.float32),
    )(x, W_emb, b_emb.reshape(1, H))

    for W_nn, b_nn, W_e, b_e in (
        (W_nn1, b_nn1, W_e1, b_e1),
        (W_nn2, b_nn2, W_e2, b_e2),
        (W_nn3, b_nn3, W_e3, b_e3),
    ):
        agg = _sc_edge_call(h, idx3, W_e[0], b_e)
        h = pl.pallas_call(
            _layer_body,
            out_shape=jax.ShapeDtypeStruct((N, H), jnp.float32),
        )(h, agg, W_nn, b_nn.reshape(1, H))

    return pl.pallas_call(
        _pool_body,
        out_shape=jax.ShapeDtypeStruct((B, 2), jnp.float32),
        scratch_shapes=[pltpu.VMEM((B, H), jnp.float32)],
    )(h, batch.reshape(N, 1), W_l1, b_l1.reshape(1, H), W_l2,
      b_l2.reshape(1, 2))


# speedup vs baseline: 1.0063x; 1.0063x over previous
import functools

import jax
import jax.numpy as jnp
from jax import lax
from jax.experimental import pallas as pl
from jax.experimental.pallas import tpu as pltpu
from jax.experimental.pallas import tpu_sc as plsc

N = 10000
E = 320000
H = 128
B = 64

NC = 2
NS = 16
LANES = 16

NW = NC * NS
CHUNK = 80
NCHUNK = 125
RBUF = 4
IBUF = 8
UNROLL = 8
MAIN_CHUNKS = 120
ROWS_PER_TILE = 624
NVREG = H // LANES
GROUPS = CHUNK // LANES


def _sc_edge_layer(h_hbm, idx_hbm, we_hbm, be_hbm, out_hbm,
                   rb0, rb1, rb2, rb3,
                   ib0, ib1, ib2, ib3, ib4, ib5, ib6, ib7,
                   we_v, be_v, agg_sp,
                   gs0, gs1, gs2, gs3, ss0, ss1, ss2, ss3,
                   is0, is1, is2, is3, is4, is5, is6, is7):
    cid = lax.axis_index("c")
    sid = lax.axis_index("s")
    wid = sid * NC + cid
    rbufs = (rb0, rb1, rb2, rb3)
    ibufs = (ib0, ib1, ib2, ib3, ib4, ib5, ib6, ib7)
    gsem = (gs0, gs1, gs2, gs3)
    ssem = (ss0, ss1, ss2, ss3)
    isem = (is0, is1, is2, is3, is4, is5, is6, is7)

    def zrow(j, _):
        zv = jnp.zeros((LANES,), jnp.float32)
        for k in range(NVREG):
            rb0[j, pl.ds(k * LANES, LANES)] = zv
        return 0
    lax.fori_loop(0, CHUNK, zrow, 0)
    for z in range(7):
        pltpu.sync_copy(rb0,
                        agg_sp.at[pl.ds(sid * ROWS_PER_TILE + z * CHUNK,
                                        CHUNK)])
    pltpu.sync_copy(rb0.at[pl.ds(0, ROWS_PER_TILE - 7 * CHUNK)],
                    agg_sp.at[pl.ds(sid * ROWS_PER_TILE + 7 * CHUNK,
                                    ROWS_PER_TILE - 7 * CHUNK)])

    @pl.when(sid == NS - 1)
    def _zero_tail():
        pltpu.sync_copy(rb0.at[pl.ds(0, N - NS * ROWS_PER_TILE)],
                        agg_sp.at[pl.ds(NS * ROWS_PER_TILE,
                                        N - NS * ROWS_PER_TILE)])

    pltpu.sync_copy(we_hbm, we_v)
    pltpu.sync_copy(be_hbm, be_v)
    wek = [we_v[pl.ds(k * LANES, LANES)] for k in range(NVREG)]
    bek = [be_v[pl.ds(k * LANES, LANES)] for k in range(NVREG)]
    plsc.subcore_barrier()

    def start_idx(i, k):
        pltpu.async_copy(idx_hbm.at[wid, i], ibufs[k], isem[k])

    def wait_idx(i, k):
        pltpu.make_async_copy(idx_hbm.at[wid, i], ibufs[k], isem[k]).wait()

    def start_gather(k, b):
        pltpu.async_copy(h_hbm.at[ibufs[k].at[0]], rbufs[b], gsem[b])

    def wait_gather(k, b):
        pltpu.make_async_copy(h_hbm.at[ibufs[k].at[0]], rbufs[b],
                              gsem[b]).wait()

    def start_scatter(k, b):
        pltpu.async_copy(rbufs[b], agg_sp.at[ibufs[k].at[1]], ssem[b],
                         add=True)

    def wait_scatter(k, b):
        pltpu.make_async_copy(rbufs[b], agg_sp.at[ibufs[k].at[1]],
                              ssem[b]).wait()

    def compute(k, b):
        def group_body(gg, _):
            ev = lax.bitcast_convert_type(
                ibufs[k][2, pl.ds(gg * LANES, LANES)], jnp.float32)
            for jj in range(LANES):
                ej = jnp.full((LANES,), ev[jj], jnp.float32)
                row = gg * LANES + jj
                for kk in range(NVREG):
                    sl = pl.ds(kk * LANES, LANES)
                    rbufs[b][row, sl] = jnp.maximum(
                        rbufs[b][row, sl] + (ej * wek[kk] + bek[kk]), 0.0)
            return 0
        lax.fori_loop(0, GROUPS, group_body, 0)

    for i in range(4):
        start_idx(i, i)
    wait_idx(0, 0)
    start_gather(0, 0)
    wait_idx(1, 1)
    start_gather(1, 1)

    def main_body(t, _):
        for u in range(UNROLL):
            i = t * UNROLL + u
            kb = u
            rb = u % RBUF

            start_idx(i + 4, (u + 4) % IBUF)
            wait_idx(i + 2, (u + 2) % IBUF)

            @pl.when(i >= 2)
            def _drain():
                wait_scatter((u - 2) % IBUF, (u + 2) % RBUF)
            start_gather((u + 2) % IBUF, (u + 2) % RBUF)
            wait_gather(kb, rb)
            compute(kb, rb)
            start_scatter(kb, rb)
        return 0
    lax.fori_loop(0, MAIN_CHUNKS // UNROLL, main_body, 0)

    for i in range(MAIN_CHUNKS, NCHUNK):
        u = i % IBUF
        rb = i % RBUF
        if i + 4 < NCHUNK:
            start_idx(i + 4, (u + 4) % IBUF)
        if i + 2 < NCHUNK:
            wait_idx(i + 2, (u + 2) % IBUF)
            wait_scatter((u - 2) % IBUF, (u + 2) % RBUF)
            start_gather((u + 2) % IBUF, (u + 2) % RBUF)
        wait_gather(u, rb)
        compute(u, rb)
        start_scatter(u, rb)
    for i in range(NCHUNK - RBUF, NCHUNK):
        wait_scatter(i % IBUF, i % RBUF)
    plsc.subcore_barrier()

    pltpu.sync_copy(agg_sp.at[pl.ds(sid * ROWS_PER_TILE, ROWS_PER_TILE)],
                    out_hbm.at[cid, pl.ds(sid * ROWS_PER_TILE, ROWS_PER_TILE)])

    @pl.when(sid == NS - 1)
    def _write_tail():
        pltpu.sync_copy(agg_sp.at[pl.ds(NS * ROWS_PER_TILE,
                                        N - NS * ROWS_PER_TILE)],
                        out_hbm.at[cid, pl.ds(NS * ROWS_PER_TILE,
                                              N - NS * ROWS_PER_TILE)])


_sc_edge_call = functools.partial(
    pl.kernel,
    out_type=jax.ShapeDtypeStruct((NC, N, H), jnp.float32),
    mesh=plsc.VectorSubcoreMesh(core_axis_name="c", subcore_axis_name="s"),
    scratch_types=(
        [pltpu.VMEM((CHUNK, H), jnp.float32)] * RBUF
        + [pltpu.VMEM((3, CHUNK), jnp.int32)] * IBUF
        + [pltpu.VMEM((H,), jnp.float32)] * 2
        + [pltpu.VMEM_SHARED((N, H), jnp.float32)]
        + [pltpu.SemaphoreType.DMA] * (RBUF + RBUF + IBUF)
    ),
)(_sc_edge_layer)


def _embed_body(x_ref, w_ref, b_ref, o_ref):
    o_ref[...] = jnp.maximum(
        jnp.dot(x_ref[...], w_ref[...], preferred_element_type=jnp.float32)
        + b_ref[...], 0.0)


def _layer_body(h_ref, agg_ref, w_ref, b_ref, o_ref):
    hh = h_ref[...] + agg_ref[0] + agg_ref[1]
    u = jnp.dot(hh, w_ref[...], preferred_element_type=jnp.float32) + b_ref[...]
    u = jnp.maximum(u, 0.0)
    u = u - jnp.mean(u, axis=0, keepdims=True)
    s = lax.rsqrt(1e-6 + jnp.mean(jnp.sum(u * u, axis=-1)))
    o_ref[...] = u * s


def _layer3_pool_body(h_ref, agg_ref, w_ref, b_ref, batch_ref, w1_ref, b1_ref,
                      w2_ref, b2_ref, o_ref, g_ref):
    hh = h_ref[...] + agg_ref[0] + agg_ref[1]
    u = jnp.dot(hh, w_ref[...], preferred_element_type=jnp.float32) + b_ref[...]
    u = jnp.maximum(u, 0.0)
    u = u - jnp.mean(u, axis=0, keepdims=True)
    s = lax.rsqrt(1e-6 + jnp.mean(jnp.sum(u * u, axis=-1)))
    h = u * s
    bvec = batch_ref[...]

    def seg(b, _):
        mask = bvec == b
        g_ref[b, :] = jnp.max(jnp.where(mask, h, -jnp.inf), axis=0)
        return 0
    lax.fori_loop(0, B, seg, 0)
    g = g_ref[...]
    u = jnp.maximum(
        jnp.dot(g, w1_ref[...], preferred_element_type=jnp.float32)
        + b1_ref[...], 0.0)
    o_ref[...] = jnp.dot(u, w2_ref[...],
                         preferred_element_type=jnp.float32) + b2_ref[...]


def kernel(x, edge_index, edge_attr, batch, W_emb, b_emb, W_nn1, b_nn1, W_e1,
           b_e1, W_nn2, b_nn2, W_e2, b_e2, W_nn3, b_nn3, W_e3, b_e3, W_l1,
           b_l1, W_l2, b_l2):
    src = edge_index[0].reshape(NW, NCHUNK, CHUNK)
    dst = edge_index[1].reshape(NW, NCHUNK, CHUNK)
    e = lax.bitcast_convert_type(edge_attr[:, 0], jnp.int32).reshape(
        NW, NCHUNK, CHUNK)
    idx3 = jnp.stack([src, dst, e], axis=2)

    h = pl.pallas_call(
        _embed_body,
        out_shape=jax.ShapeDtypeStruct((N, H), jnp.float32),
    )(x, W_emb, b_emb.reshape(1, H))

    for W_nn, b_nn, W_e, b_e in (
        (W_nn1, b_nn1, W_e1, b_e1),
        (W_nn2, b_nn2, W_e2, b_e2),
    ):
        agg = _sc_edge_call(h, idx3, W_e[0], b_e)
        h = pl.pallas_call(
            _layer_body,
            out_shape=jax.ShapeDtypeStruct((N, H), jnp.float32),
        )(h, agg, W_nn, b_nn.reshape(1, H))

    agg = _sc_edge_call(h, idx3, W_e3[0], b_e3)
    return pl.pallas_call(
        _layer3_pool_body,
        out_shape=jax.ShapeDtypeStruct((B, 2), jnp.float32),
        scratch_shapes=[pltpu.VMEM((B, H), jnp.float32)],
    )(h, agg, W_nn3, b_nn3.reshape(1, H), batch.reshape(N, 1), W_l1,
      b_l1.reshape(1, H), W_l2, b_l2.reshape(1, 2))


# speedup vs baseline: 1.2067x; 1.1991x over previous
import functools

import jax
import jax.numpy as jnp
from jax import lax
from jax.experimental import pallas as pl
from jax.experimental.pallas import tpu as pltpu
from jax.experimental.pallas import tpu_sc as plsc

N = 10000
E = 320000
H = 128
B = 64

NC = 2
NS = 16
LANES = 16

NW = NC * NS
CHUNK = 80
NCHUNK = 125
EDGES_PER_W = NCHUNK * CHUNK
RBUF = 4
IBUF = 8
UNROLL = 8
MAIN_CHUNKS = 120
ROWS_PER_TILE = 624
NVREG = H // LANES
GROUPS = CHUNK // LANES
CHS = 64


def _sc_edge_layer(h_hbm, src_hbm, dst_hbm, e_hbm, we_hbm, be_hbm, out_hbm,
                   rb0, rb1, rb2, rb3,
                   ib0, ib1, ib2, ib3, ib4, ib5, ib6, ib7,
                   eb0, eb1, eb2, eb3, eb4, eb5, eb6, eb7,
                   we_v, be_v, agg_sp,
                   gs0, gs1, gs2, gs3, ss0, ss1, ss2, ss3,
                   is0, is1, is2, is3, is4, is5, is6, is7):
    cid = lax.axis_index("c")
    sid = lax.axis_index("s")
    wid = sid * NC + cid
    rbufs = (rb0, rb1, rb2, rb3)
    ibufs = (ib0, ib1, ib2, ib3, ib4, ib5, ib6, ib7)
    ebufs = (eb0, eb1, eb2, eb3, eb4, eb5, eb6, eb7)
    gsem = (gs0, gs1, gs2, gs3)
    ssem = (ss0, ss1, ss2, ss3)
    isem = (is0, is1, is2, is3, is4, is5, is6, is7)

    def zrow(j, _):
        zv = jnp.zeros((LANES,), jnp.float32)
        for k in range(NVREG):
            rb0[j, pl.ds(k * LANES, LANES)] = zv
        return 0
    lax.fori_loop(0, CHUNK, zrow, 0)
    for z in range(7):
        pltpu.sync_copy(rb0,
                        agg_sp.at[pl.ds(sid * ROWS_PER_TILE + z * CHUNK,
                                        CHUNK)])
    pltpu.sync_copy(rb0.at[pl.ds(0, ROWS_PER_TILE - 7 * CHUNK)],
                    agg_sp.at[pl.ds(sid * ROWS_PER_TILE + 7 * CHUNK,
                                    ROWS_PER_TILE - 7 * CHUNK)])

    @pl.when(sid == NS - 1)
    def _zero_tail():
        pltpu.sync_copy(rb0.at[pl.ds(0, N - NS * ROWS_PER_TILE)],
                        agg_sp.at[pl.ds(NS * ROWS_PER_TILE,
                                        N - NS * ROWS_PER_TILE)])

    pltpu.sync_copy(we_hbm, we_v)
    pltpu.sync_copy(be_hbm, be_v)
    wek = [we_v[pl.ds(k * LANES, LANES)] for k in range(NVREG)]
    bek = [be_v[pl.ds(k * LANES, LANES)] for k in range(NVREG)]
    plsc.subcore_barrier()

    def start_idx(i, k):
        base = wid * EDGES_PER_W + i * CHUNK
        pltpu.async_copy(src_hbm.at[pl.ds(base, CHUNK)], ibufs[k].at[0],
                         isem[k])
        pltpu.async_copy(dst_hbm.at[pl.ds(base, CHUNK)], ibufs[k].at[1],
                         isem[k])
        pltpu.async_copy(e_hbm.at[pl.ds(base, CHUNK)], ebufs[k], isem[k])

    def wait_idx(i, k):
        base = wid * EDGES_PER_W + i * CHUNK
        pltpu.make_async_copy(src_hbm.at[pl.ds(base, CHUNK)], ibufs[k].at[0],
                              isem[k]).wait()
        pltpu.make_async_copy(dst_hbm.at[pl.ds(base, CHUNK)], ibufs[k].at[1],
                              isem[k]).wait()
        pltpu.make_async_copy(e_hbm.at[pl.ds(base, CHUNK)], ebufs[k],
                              isem[k]).wait()

    def start_gather(k, b):
        pltpu.async_copy(h_hbm.at[ibufs[k].at[0]], rbufs[b], gsem[b])

    def wait_gather(k, b):
        pltpu.make_async_copy(h_hbm.at[ibufs[k].at[0]], rbufs[b],
                              gsem[b]).wait()

    def start_scatter(k, b):
        pltpu.async_copy(rbufs[b], agg_sp.at[ibufs[k].at[1]], ssem[b],
                         add=True)

    def wait_scatter(k, b):
        pltpu.make_async_copy(rbufs[b], agg_sp.at[ibufs[k].at[1]],
                              ssem[b]).wait()

    def compute(k, b):
        def group_body(gg, _):
            ev = ebufs[k][pl.ds(gg * LANES, LANES)]
            for jj in range(LANES):
                ej = jnp.full((LANES,), ev[jj], jnp.float32)
                row = gg * LANES + jj
                for kk in range(NVREG):
                    sl = pl.ds(kk * LANES, LANES)
                    rbufs[b][row, sl] = jnp.maximum(
                        rbufs[b][row, sl] + (ej * wek[kk] + bek[kk]), 0.0)
            return 0
        lax.fori_loop(0, GROUPS, group_body, 0)

    for i in range(4):
        start_idx(i, i)
    wait_idx(0, 0)
    start_gather(0, 0)
    wait_idx(1, 1)
    start_gather(1, 1)

    def main_body(t, _):
        for u in range(UNROLL):
            i = t * UNROLL + u
            kb = u
            rb = u % RBUF

            @pl.when(i + 4 < NCHUNK)
            def _fetch():
                start_idx(i + 4, (u + 4) % IBUF)

            @pl.when(i + 2 < NCHUNK)
            def _prefetch():
                wait_idx(i + 2, (u + 2) % IBUF)

                @pl.when(i >= 2)
                def _drain():
                    wait_scatter((u - 2) % IBUF, (u + 2) % RBUF)
                start_gather((u + 2) % IBUF, (u + 2) % RBUF)

            @pl.when(i < NCHUNK)
            def _work():
                wait_gather(kb, rb)
                compute(kb, rb)
                start_scatter(kb, rb)
        return 0
    lax.fori_loop(0, (NCHUNK + UNROLL - 1) // UNROLL, main_body, 0)

    for i in range(NCHUNK - RBUF, NCHUNK):
        wait_scatter(i % IBUF, i % RBUF)
    plsc.subcore_barrier()

    pltpu.sync_copy(agg_sp.at[pl.ds(sid * ROWS_PER_TILE, ROWS_PER_TILE)],
                    out_hbm.at[cid, pl.ds(sid * ROWS_PER_TILE, ROWS_PER_TILE)])

    @pl.when(sid == NS - 1)
    def _write_tail():
        pltpu.sync_copy(agg_sp.at[pl.ds(NS * ROWS_PER_TILE,
                                        N - NS * ROWS_PER_TILE)],
                        out_hbm.at[cid, pl.ds(NS * ROWS_PER_TILE,
                                              N - NS * ROWS_PER_TILE)])


_sc_edge_call = functools.partial(
    pl.kernel,
    out_type=jax.ShapeDtypeStruct((NC, N, H), jnp.float32),
    mesh=plsc.VectorSubcoreMesh(core_axis_name="c", subcore_axis_name="s"),
    scratch_types=(
        [pltpu.VMEM((CHUNK, H), jnp.float32)] * RBUF
        + [pltpu.VMEM((2, CHUNK), jnp.int32)] * IBUF
        + [pltpu.VMEM((CHUNK,), jnp.float32)] * IBUF
        + [pltpu.VMEM((H,), jnp.float32)] * 2
        + [pltpu.VMEM_SHARED((N, H), jnp.float32)]
        + [pltpu.SemaphoreType.DMA] * (RBUF + RBUF + IBUF)
    ),
)(_sc_edge_layer)


def _embed_body(x_ref, w_ref, b_ref, o_ref):
    o_ref[...] = jnp.maximum(
        jnp.dot(x_ref[...], w_ref[...], preferred_element_type=jnp.float32)
        + b_ref[...], 0.0)


def _layer_body(h_ref, agg_ref, w_ref, b_ref, o_ref):
    hh = h_ref[...] + agg_ref[0] + agg_ref[1]
    u = jnp.dot(hh, w_ref[...], preferred_element_type=jnp.float32) + b_ref[...]
    u = jnp.maximum(u, 0.0)
    u = u - jnp.mean(u, axis=0, keepdims=True)
    s = lax.rsqrt(1e-6 + jnp.mean(jnp.sum(u * u, axis=-1)))
    o_ref[...] = u * s


def _layer3_pool_body(h_ref, agg_ref, w_ref, b_ref, starts_ref, ends_ref,
                      w1_ref, b1_ref, w2_ref, b2_ref, o_ref, h_scr, g_ref):
    hh = h_ref[...] + agg_ref[0] + agg_ref[1]
    u = jnp.dot(hh, w_ref[...], preferred_element_type=jnp.float32) + b_ref[...]
    u = jnp.maximum(u, 0.0)
    u = u - jnp.mean(u, axis=0, keepdims=True)
    s = lax.rsqrt(1e-6 + jnp.mean(jnp.sum(u * u, axis=-1)))
    h_scr[...] = u * s

    iota_c = lax.broadcasted_iota(jnp.int32, (CHS, 1), 0)
    neg = jnp.full((CHS, H), -jnp.inf, jnp.float32)

    def seg(b, _):
        s0 = starts_ref[b]
        e0 = ends_ref[b]
        blo = s0 // CHS
        nblk = jnp.where(e0 > s0, (e0 - 1) // CHS - blo + 1, 0)

        def blk(t, acc):
            st = jnp.minimum((blo + t) * CHS, N - CHS)
            rows = h_scr[pl.ds(st, CHS), :]
            idx = st + iota_c
            m = jnp.logical_and(idx >= s0, idx < e0)
            return jnp.maximum(acc, jnp.where(m, rows, -jnp.inf))
        acc = lax.fori_loop(0, nblk, blk, neg)
        g_ref[b, :] = jnp.max(acc, axis=0)
        return 0
    lax.fori_loop(0, B, seg, 0)

    g = g_ref[...]
    u = jnp.maximum(
        jnp.dot(g, w1_ref[...], preferred_element_type=jnp.float32)
        + b1_ref[...], 0.0)
    o_ref[...] = jnp.dot(u, w2_ref[...],
                         preferred_element_type=jnp.float32) + b2_ref[...]


def kernel(x, edge_index, edge_attr, batch, W_emb, b_emb, W_nn1, b_nn1, W_e1,
           b_e1, W_nn2, b_nn2, W_e2, b_e2, W_nn3, b_nn3, W_e3, b_e3, W_l1,
           b_l1, W_l2, b_l2):
    src = edge_index[0]
    dst = edge_index[1]
    e = edge_attr[:, 0]
    seg_ids = jnp.arange(B, dtype=batch.dtype)
    starts = jnp.searchsorted(batch, seg_ids, side="left").astype(jnp.int32)
    ends = jnp.searchsorted(batch, seg_ids, side="right").astype(jnp.int32)

    h = pl.pallas_call(
        _embed_body,
        out_shape=jax.ShapeDtypeStruct((N, H), jnp.float32),
    )(x, W_emb, b_emb.reshape(1, H))

    for W_nn, b_nn, W_e, b_e in (
        (W_nn1, b_nn1, W_e1, b_e1),
        (W_nn2, b_nn2, W_e2, b_e2),
    ):
        agg = _sc_edge_call(h, src, dst, e, W_e[0], b_e)
        h = pl.pallas_call(
            _layer_body,
            out_shape=jax.ShapeDtypeStruct((N, H), jnp.float32),
        )(h, agg, W_nn, b_nn.reshape(1, H))

    agg = _sc_edge_call(h, src, dst, e, W_e3[0], b_e3)
    vspec = pl.BlockSpec(memory_space=pltpu.VMEM)
    sspec = pl.BlockSpec(memory_space=pltpu.SMEM)
    return pl.pallas_call(
        _layer3_pool_body,
        out_shape=jax.ShapeDtypeStruct((B, 2), jnp.float32),
        in_specs=[vspec, vspec, vspec, vspec, sspec, sspec,
                  vspec, vspec, vspec, vspec],
        scratch_shapes=[pltpu.VMEM((N, H), jnp.float32),
                        pltpu.VMEM((B, H), jnp.float32)],
    )(h, agg, W_nn3, b_nn3.reshape(1, H), starts, ends, W_l1,
      b_l1.reshape(1, H), W_l2, b_l2.reshape(1, 2))
